# baseline (speedup 1.0000x reference)
import jax
import jax.numpy as jnp
from jax import lax
from jax.experimental import pallas as pl
from jax.experimental.pallas import tpu as pltpu
from jax.experimental.pallas import tpu_sc as plsc

_N_ATOMS = 819200
_N_FEATURES = 64
_TABLE_ROWS = 83
_NC = 2
_NS = 16
_NUM_WORKERS = _NC * _NS
_B_PER_W = _N_ATOMS // _NUM_WORKERS
_CHUNK = 640
_N_CHUNKS = _B_PER_W // _CHUNK
_L = 16


def _emb_body(x_hbm, table_hbm, out_hbm, table_v, idx_v, rows0, rows1,
              semc0, semc1):
    cid = lax.axis_index("c")
    sid = lax.axis_index("s")
    wid = sid * _NC + cid
    base = wid * _B_PER_W

    pltpu.sync_copy(table_hbm, table_v)
    pltpu.sync_copy(x_hbm.at[pl.ds(base, _B_PER_W)], idx_v)

    rows = (rows0, rows1)
    semc = (semc0, semc1)

    lane64 = lax.iota(jnp.int32, _L) * _N_FEATURES

    def compute(i, b):
        rbuf = rows[b]

        def group(g, carry):
            xs = idx_v[pl.ds(i * _CHUNK + g * _L, _L)] * _N_FEATURES
            dst = lane64 + g * (_L * _N_FEATURES)
            for c in range(_N_FEATURES):
                v = plsc.load_gather(table_v, [xs + c])
                plsc.store_scatter(rbuf, [dst + c], v)
            return carry

        lax.fori_loop(0, _CHUNK // _L, group, 0)

    def put(i, b):
        return pltpu.async_copy(
            rows[b],
            out_hbm.at[pl.ds((base + i * _CHUNK) * _N_FEATURES,
                             _CHUNK * _N_FEATURES)],
            semc[b])

    compute(0, 0)
    put(0, 0)
    compute(1, 1)
    put(1, 1)

    def pair(j, carry):
        for b in range(2):
            i = 2 * j + b
            pltpu.make_async_copy(
                rows[b], out_hbm.at[pl.ds(0, _CHUNK * _N_FEATURES)],
                semc[b]).wait()
            compute(i, b)
            put(i, b)
        return carry

    lax.fori_loop(1, _N_CHUNKS // 2, pair, 0)

    pltpu.make_async_copy(rows0, out_hbm.at[pl.ds(0, _CHUNK * _N_FEATURES)],
                          semc0).wait()
    pltpu.make_async_copy(rows1, out_hbm.at[pl.ds(0, _CHUNK * _N_FEATURES)],
                          semc1).wait()


@jax.jit
def kernel(x, embedding_list):
    run = pl.kernel(
        _emb_body,
        out_type=jax.ShapeDtypeStruct((_N_ATOMS * _N_FEATURES,), jnp.float32),
        mesh=plsc.VectorSubcoreMesh(core_axis_name="c", subcore_axis_name="s"),
        scratch_types=[
            pltpu.VMEM((_TABLE_ROWS * _N_FEATURES,), jnp.float32),
            pltpu.VMEM((_B_PER_W,), jnp.int32),
            pltpu.VMEM((_CHUNK * _N_FEATURES,), jnp.float32),
            pltpu.VMEM((_CHUNK * _N_FEATURES,), jnp.float32),
            pltpu.SemaphoreType.DMA,
            pltpu.SemaphoreType.DMA,
        ],
        compiler_params=pltpu.CompilerParams(use_tc_tiling_on_sc=False,
                                             needs_layout_passes=False),
    )
    flat = run(x, embedding_list.reshape(-1))
    return flat.reshape(_N_ATOMS, _N_FEATURES)


# speedup vs baseline: 2.7012x; 2.7012x over previous
import jax
import jax.numpy as jnp
from jax import lax
from jax.experimental import pallas as pl
from jax.experimental.pallas import tpu as pltpu
from jax.experimental.pallas import tpu_sc as plsc

_N_ATOMS = 819200
_N_FEATURES = 64
_TABLE_ROWS = 83
_NC = 2
_NS = 16
_NUM_WORKERS = _NC * _NS
_B_PER_W = _N_ATOMS // _NUM_WORKERS
_CHUNK = 640
_N_CHUNKS = _B_PER_W // _CHUNK
_L = 16


def _emb_body(x_hbm, table_hbm, out_hbm, table_v, idx_v, rows0, rows1,
              semc0, semc1):
    cid = lax.axis_index("c")
    sid = lax.axis_index("s")
    wid = sid * _NC + cid
    base = wid * _B_PER_W

    pltpu.sync_copy(table_hbm, table_v)
    pltpu.sync_copy(x_hbm.at[pl.ds(base, _B_PER_W)], idx_v)

    rows = (rows0, rows1)
    semc = (semc0, semc1)

    def compute(i, b):
        rbuf = rows[b]

        def group(g, carry):
            r0 = i * _CHUNK + g * _L
            xv = idx_v[pl.ds(r0, _L)] * _N_FEATURES
            for rr in range(_L):
                src = xv[rr]
                dst = (g * _L + rr) * _N_FEATURES
                for c in range(_N_FEATURES // _L):
                    rbuf[pl.ds(dst + c * _L, _L)] = (
                        table_v[pl.ds(src + c * _L, _L)])
            return carry

        lax.fori_loop(0, _CHUNK // _L, group, 0)

    def put(i, b):
        return pltpu.async_copy(
            rows[b],
            out_hbm.at[pl.ds((base + i * _CHUNK) * _N_FEATURES,
                             _CHUNK * _N_FEATURES)],
            semc[b])

    compute(0, 0)
    put(0, 0)
    compute(1, 1)
    put(1, 1)

    def pair(j, carry):
        for b in range(2):
            i = 2 * j + b
            pltpu.make_async_copy(
                rows[b], out_hbm.at[pl.ds(0, _CHUNK * _N_FEATURES)],
                semc[b]).wait()
            compute(i, b)
            put(i, b)
        return carry

    lax.fori_loop(1, _N_CHUNKS // 2, pair, 0)

    pltpu.make_async_copy(rows0, out_hbm.at[pl.ds(0, _CHUNK * _N_FEATURES)],
                          semc0).wait()
    pltpu.make_async_copy(rows1, out_hbm.at[pl.ds(0, _CHUNK * _N_FEATURES)],
                          semc1).wait()


@jax.jit
def kernel(x, embedding_list):
    run = pl.kernel(
        _emb_body,
        out_type=jax.ShapeDtypeStruct((_N_ATOMS * _N_FEATURES,), jnp.float32),
        mesh=plsc.VectorSubcoreMesh(core_axis_name="c", subcore_axis_name="s"),
        scratch_types=[
            pltpu.VMEM((_TABLE_ROWS * _N_FEATURES,), jnp.float32),
            pltpu.VMEM((_B_PER_W,), jnp.int32),
            pltpu.VMEM((_CHUNK * _N_FEATURES,), jnp.float32),
            pltpu.VMEM((_CHUNK * _N_FEATURES,), jnp.float32),
            pltpu.SemaphoreType.DMA,
            pltpu.SemaphoreType.DMA,
        ],
        compiler_params=pltpu.CompilerParams(use_tc_tiling_on_sc=False,
                                             needs_layout_passes=False),
    )
    flat = run(x, embedding_list.reshape(-1))
    return flat.reshape(_N_ATOMS, _N_FEATURES)


# speedup vs baseline: 4.0470x; 1.4982x over previous
import jax
import jax.numpy as jnp
from jax import lax
from jax.experimental import pallas as pl
from jax.experimental.pallas import tpu as pltpu
from jax.experimental.pallas import tpu_sc as plsc

_N_ATOMS = 819200
_N_FEATURES = 64
_TABLE_ROWS = 83
_NC = 2
_NS = 16
_NUM_WORKERS = _NC * _NS
_B_PER_W = _N_ATOMS // _NUM_WORKERS
_RPG = 128
_GPC = 5
_CHUNK = _RPG * _GPC
_N_CHUNKS = _B_PER_W // _CHUNK
_IDX_ROWS = _B_PER_W // _RPG


def _emb_body(x_hbm, table_hbm, out_hbm, table_sh, idx_v, rows0, rows1,
              gsem0, gsem1, wsem0, wsem1):
    cid = lax.axis_index("c")
    sid = lax.axis_index("s")
    wid = sid * _NC + cid
    row_base = wid * _B_PER_W

    @pl.when(sid == 0)
    def _stage_table():
        pltpu.sync_copy(table_hbm, rows0.at[pl.ds(0, _TABLE_ROWS)])
        pltpu.sync_copy(rows0.at[pl.ds(0, _TABLE_ROWS)], table_sh)

    pltpu.sync_copy(x_hbm.at[pl.ds(wid * _IDX_ROWS, _IDX_ROWS)], idx_v)
    plsc.subcore_barrier()

    rows = (rows0, rows1)
    gsem = (gsem0, gsem1)
    wsem = (wsem0, wsem1)

    def gathers(i, b):
        for k in range(_GPC):
            pltpu.async_copy(
                table_sh.at[idx_v.at[i * _GPC + k]],
                rows[b].at[pl.ds(k * _RPG, _RPG)],
                gsem[b])
        pltpu.make_async_copy(out_hbm.at[pl.ds(0, _CHUNK)], rows[b],
                              gsem[b]).wait()

    def put(i, b):
        pltpu.async_copy(rows[b],
                         out_hbm.at[pl.ds(row_base + i * _CHUNK, _CHUNK)],
                         wsem[b])

    def wait_put(b):
        pltpu.make_async_copy(rows[b], out_hbm.at[pl.ds(0, _CHUNK)],
                              wsem[b]).wait()

    gathers(0, 0)
    put(0, 0)
    gathers(1, 1)
    put(1, 1)

    def pair(j, carry):
        for b in range(2):
            i = 2 * j + b
            wait_put(b)
            gathers(i, b)
            put(i, b)
        return carry

    lax.fori_loop(1, _N_CHUNKS // 2, pair, 0)

    wait_put(0)
    wait_put(1)


@jax.jit
def kernel(x, embedding_list):
    run = pl.kernel(
        _emb_body,
        out_type=jax.ShapeDtypeStruct((_N_ATOMS, _N_FEATURES), jnp.float32),
        mesh=plsc.VectorSubcoreMesh(core_axis_name="c", subcore_axis_name="s"),
        scratch_types=[
            pltpu.VMEM_SHARED((_TABLE_ROWS, _N_FEATURES), jnp.float32),
            pltpu.VMEM((_IDX_ROWS, _RPG), jnp.int32),
            pltpu.VMEM((_CHUNK, _N_FEATURES), jnp.float32),
            pltpu.VMEM((_CHUNK, _N_FEATURES), jnp.float32),
            pltpu.SemaphoreType.DMA,
            pltpu.SemaphoreType.DMA,
            pltpu.SemaphoreType.DMA,
            pltpu.SemaphoreType.DMA,
        ],
        compiler_params=pltpu.CompilerParams(use_tc_tiling_on_sc=False,
                                             needs_layout_passes=False),
    )
    return run(x.reshape(-1, _RPG), embedding_list)
